# Initial kernel scaffold; baseline (speedup 1.0000x reference)
import jax
import jax.numpy as jnp
from jax.experimental import pallas as pl


def kernel(x):
    raise NotImplementedError("write your pallas kernel here")


# speedup vs baseline: 35.1935x; 35.1935x over previous
import jax
import jax.numpy as jnp
from jax.experimental import pallas as pl

_B, _C, _H, _W = 4, 4, 64, 64


def _pre_kernel(x_ref, o_ref):
    x = x_ref[...]

    row = jax.lax.broadcasted_iota(jnp.int32, (_H, _H), 0)
    col = jax.lax.broadcasted_iota(jnp.int32, (_H, _H), 1)
    d2 = ((row - col) * (row - col)).astype(jnp.float32)
    g1 = jnp.exp(d2 * (-0.5))
    g05 = jnp.exp(d2 * (-2.0))

    def gauss_map(chan, g):
        raw = x[:, chan, :, :]
        mask = (raw > 0).astype(jnp.float32)
        maps = []
        for b in range(_B):
            m = mask[b]
            t = jax.lax.dot(g, m, preferred_element_type=jnp.float32)
            maps.append(jax.lax.dot(t, g, preferred_element_type=jnp.float32))
        batched = jnp.stack(maps, axis=0)
        maxv = jnp.max(batched.reshape(_B, _H * _W), axis=1).reshape(_B, 1, 1)
        maxv = jnp.where(maxv == 0, 1.0, maxv)
        has_pos = jnp.sum(mask) > 0
        return jnp.where(has_pos, batched / maxv, raw)

    o0 = x[:, 0, :, :]
    o1 = gauss_map(2, g1)
    o2 = gauss_map(1, g1)
    o3 = gauss_map(3, g05)
    o4 = o1 * o2
    o_ref[...] = jnp.stack([o0, o1, o2, o3, o4], axis=1)


def kernel(x):
    return pl.pallas_call(
        _pre_kernel,
        out_shape=jax.ShapeDtypeStruct((_B, 5, _H, _W), jnp.float32),
    )(x)
